# Initial kernel scaffold; baseline (speedup 1.0000x reference)
import jax
import jax.numpy as jnp
from jax.experimental import pallas as pl


def kernel(node_pos, node_sizes, full_edge_index, batch, full_edge_attr):
    raise NotImplementedError("write your pallas kernel here")


# speedup vs baseline: 19.5656x; 19.5656x over previous
import functools

import jax
import jax.numpy as jnp
from jax import lax
from jax.experimental import pallas as pl
from jax.experimental.pallas import tpu as pltpu
from jax.experimental.pallas import tpu_sc as plsc

N = 100000
E = 6400000
G = 128
NW = 32
CH = 2048
SUB = 16
NCHUNK = E // CH
NPAD = NW * 3136
NNODE_IT = 3136 // 16

_L16 = 16


def _iota16():
    return lax.iota(jnp.int32, _L16)


def _rsqrt(x):
    bits = plsc.bitcast(x, jnp.int32)
    y = plsc.bitcast(jnp.full((_L16,), 0x5F3759DF, jnp.int32) - jnp.right_shift(bits, 1), jnp.float32)
    half = x * 0.5
    for _ in range(3):
        y = y * (1.5 - half * y * y)
    return y


def _recip(x):
    bits = plsc.bitcast(x, jnp.int32)
    y = plsc.bitcast(jnp.full((_L16,), 0x7EF311C3, jnp.int32) - bits, jnp.float32)
    for _ in range(3):
        y = y * (2.0 - x * y)
    return y


def _wid():
    return lax.axis_index("s") * 2 + lax.axis_index("c")


def _nchunks(wid):
    return (NCHUNK - 1 - wid) // NW + 1


def _pass1_body(start_hbm, end_hbm, attr_hbm, table_hbm, batch_hbm,
                rec_hbm, sums_hbm,
                s_idx, e_idx, attrb, rows_s, rows_e,
                r0, r1, r2, r3, r4, r5,
                acc_num, acc_den, acc_ov, acc_cnt, acc_gsz,
                nbuf, stage, sem):
    wid = _wid()
    lane = _iota16()
    lane128 = lane * G
    zeros16 = jnp.zeros((_L16,), jnp.float32)
    ones16 = jnp.ones((_L16,), jnp.float32)
    c0 = jnp.zeros((_L16,), jnp.int32)
    c1 = jnp.full((_L16,), 1, jnp.int32)
    c2 = jnp.full((_L16,), 2, jnp.int32)
    c3 = jnp.full((_L16,), 3, jnp.int32)
    c4 = jnp.full((_L16,), 4, jnp.int32)

    @pl.loop(0, G)
    def _zero(i):
        sl = pl.ds(i * _L16, _L16)
        acc_num[sl] = zeros16
        acc_den[sl] = zeros16
        acc_ov[sl] = zeros16
        acc_cnt[sl] = zeros16
        acc_gsz[sl] = zeros16

    pltpu.sync_copy(batch_hbm.at[pl.ds(wid * 3136, 3136)], nbuf)

    @pl.loop(0, NNODE_IT)
    def _hist(t):
        v = nbuf[pl.ds(t * _L16, _L16)]
        gid = wid * 3136 + t * _L16 + lane
        mask = gid < N
        plsc.addupdate_scatter(acc_gsz, [lane128 + v], ones16, mask=mask)

    @pl.loop(0, _nchunks(wid))
    def _chunk(j):
        chunk = wid + NW * j
        off = chunk * CH
        row0 = chunk * SUB
        pltpu.sync_copy(start_hbm.at[pl.ds(row0, SUB)], s_idx)
        pltpu.sync_copy(end_hbm.at[pl.ds(row0, SUB)], e_idx)
        pltpu.sync_copy(attr_hbm.at[pl.ds(off, CH)], attrb)
        cps = []
        for k in range(SUB):
            cps.append(pltpu.async_copy(table_hbm.at[s_idx.at[k]], rows_s.at[k], sem))
            cps.append(pltpu.async_copy(table_hbm.at[e_idx.at[k]], rows_e.at[k], sem))
        for cp in cps:
            cp.wait()

        @pl.loop(0, CH // _L16)
        def _grp(g):
            base = g * _L16
            ev = base + lane
            jv = jnp.right_shift(ev, 7)
            rv = jnp.bitwise_and(ev, 127)
            px_s = plsc.load_gather(rows_s, [jv, rv, c0])
            py_s = plsc.load_gather(rows_s, [jv, rv, c1])
            sx_s = plsc.load_gather(rows_s, [jv, rv, c2])
            sy_s = plsc.load_gather(rows_s, [jv, rv, c3])
            b_s = plsc.bitcast(plsc.load_gather(rows_s, [jv, rv, c4]), jnp.int32)
            px_e = plsc.load_gather(rows_e, [jv, rv, c0])
            py_e = plsc.load_gather(rows_e, [jv, rv, c1])
            sx_e = plsc.load_gather(rows_e, [jv, rv, c2])
            sy_e = plsc.load_gather(rows_e, [jv, rv, c3])
            b_e = plsc.bitcast(plsc.load_gather(rows_e, [jv, rv, c4]), jnp.int32)
            d = plsc.load_gather(attrb, [ev, c0])

            invd = _recip(d)
            dx = px_s - px_e
            dy = py_s - py_e
            q = dx * dx + dy * dy
            eu = q * _rsqrt(q)
            r = eu * invd
            binv = lane128 + b_s
            plsc.addupdate_scatter(acc_num, [binv], r * r)
            plsc.addupdate_scatter(acc_den, [binv], r)

            ox = jnp.maximum((sx_s + sx_e) * 0.5 - jnp.abs(dx), 0.0)
            oy = jnp.maximum((sy_s + sy_e) * 0.5 - jnp.abs(dy), 0.0)
            tot = sx_s + sy_s + sx_e + sy_e
            nov = ox * oy * _recip(tot)
            plsc.addupdate_scatter(acc_ov, [binv], nov)
            plsc.addupdate_scatter(acc_cnt, [binv], ones16)

            sl = pl.ds(base, _L16)
            r0[sl] = px_s
            r1[sl] = py_s
            r2[sl] = px_e
            r3[sl] = py_e
            r4[sl] = plsc.bitcast(jnp.left_shift(b_s, 7) + b_e, jnp.float32)
            r5[sl] = invd

        for q_i, buf in enumerate((r0, r1, r2, r3, r4, r5)):
            pltpu.sync_copy(buf, rec_hbm.at[q_i, pl.ds(off, CH)])

    for q_i, acc in enumerate((acc_num, acc_den, acc_ov, acc_cnt, acc_gsz)):
        for blk in range(G // _L16):
            tot = acc[pl.ds(blk * _L16, _L16)]
            for l in range(1, _L16):
                tot = tot + acc[pl.ds(l * G + blk * _L16, _L16)]
            stage[pl.ds(q_i * G + blk * _L16, _L16)] = tot
    pltpu.sync_copy(stage, sums_hbm.at[pl.ds(wid * 640, 640)])


def _pass2_body(rec_hbm, sums_hbm, stress_hbm,
                sbuf, invs, b0, b1, b2, b3, b4, b5, acc_st, stage, sem):
    wid = _wid()
    lane = _iota16()
    lane128 = lane * G
    zeros16 = jnp.zeros((_L16,), jnp.float32)

    @pl.loop(0, G)
    def _zero(i):
        acc_st[pl.ds(i * _L16, _L16)] = zeros16

    pltpu.sync_copy(sums_hbm, sbuf)
    for blk in range(G // _L16):
        ntot = sbuf[pl.ds(blk * _L16, _L16)]
        dtot = sbuf[pl.ds(G + blk * _L16, _L16)]
        for w in range(1, NW):
            ntot = ntot + sbuf[pl.ds(w * 640 + blk * _L16, _L16)]
            dtot = dtot + sbuf[pl.ds(w * 640 + G + blk * _L16, _L16)]
        invs[pl.ds(blk * _L16, _L16)] = dtot * _recip(ntot)

    @pl.loop(0, _nchunks(wid))
    def _chunk(j):
        chunk = wid + NW * j
        off = chunk * CH
        bufs = (b0, b1, b2, b3, b4, b5)
        cps = [pltpu.async_copy(rec_hbm.at[q_i, pl.ds(off, CH)], bufs[q_i], sem)
               for q_i in range(6)]
        for cp in cps:
            cp.wait()

        @pl.loop(0, CH // _L16)
        def _grp(g):
            sl = pl.ds(g * _L16, _L16)
            px_s = b0[sl]
            py_s = b1[sl]
            px_e = b2[sl]
            py_e = b3[sl]
            bp = plsc.bitcast(b4[sl], jnp.int32)
            invd = b5[sl]
            b_s = jnp.right_shift(bp, 7)
            b_e = jnp.bitwise_and(bp, 127)
            u = plsc.load_gather(invs, [b_s])
            v = plsc.load_gather(invs, [b_e])
            ddx = px_s * u - px_e * v
            ddy = py_s * u - py_e * v
            q2 = ddx * ddx + ddy * ddy
            eu2 = q2 * _rsqrt(q2)
            t = eu2 * invd - 1.0
            plsc.addupdate_scatter(acc_st, [lane128 + b_s], t * t)

    for blk in range(G // _L16):
        tot = acc_st[pl.ds(blk * _L16, _L16)]
        for l in range(1, _L16):
            tot = tot + acc_st[pl.ds(l * G + blk * _L16, _L16)]
        stage[pl.ds(blk * _L16, _L16)] = tot
    pltpu.sync_copy(stage, stress_hbm.at[pl.ds(wid * G, G)])


def _fin_body(sums_ref, stress_ref, o_ref):
    s = sums_ref[:]
    ov = jnp.sum(s[:, 2 * G:3 * G], axis=0)
    cnt = jnp.sum(s[:, 3 * G:4 * G], axis=0)
    gsz = jnp.sum(s[:, 4 * G:5 * G], axis=0)
    st = jnp.sum(stress_ref[:], axis=0)
    combined = st / (gsz * gsz) + ov / cnt
    o_ref[:, :] = jnp.mean(combined)[None, None]


@jax.jit
def kernel(node_pos, node_sizes, full_edge_index, batch, full_edge_attr):
    table = jnp.concatenate(
        [node_pos, node_sizes,
         lax.bitcast_convert_type(batch, jnp.float32)[:, None],
         jnp.zeros((N, 11), jnp.float32)], axis=1)
    start2d = full_edge_index[0].reshape(E // 128, 128)
    end2d = full_edge_index[1].reshape(E // 128, 128)
    batch_pad = jnp.concatenate([batch, jnp.zeros((NPAD - N,), jnp.int32)])

    mesh = plsc.VectorSubcoreMesh(core_axis_name="c", subcore_axis_name="s")

    p1 = pl.kernel(
        _pass1_body,
        out_type=[jax.ShapeDtypeStruct((6, E), jnp.float32),
                  jax.ShapeDtypeStruct((NW * 640,), jnp.float32)],
        mesh=mesh,
        compiler_params=pltpu.CompilerParams(needs_layout_passes=False,
                                             use_tc_tiling_on_sc=False),
        scratch_types=[
            pltpu.VMEM((SUB, 128), jnp.int32),
            pltpu.VMEM((SUB, 128), jnp.int32),
            pltpu.VMEM((CH, 4), jnp.float32),
            pltpu.VMEM((SUB, 128, 16), jnp.float32),
            pltpu.VMEM((SUB, 128, 16), jnp.float32),
        ] + [pltpu.VMEM((CH,), jnp.float32)] * 6
          + [pltpu.VMEM((_L16 * G,), jnp.float32)] * 5
          + [
            pltpu.VMEM((3136,), jnp.int32),
            pltpu.VMEM((640,), jnp.float32),
            pltpu.SemaphoreType.DMA,
        ],
    )
    rec, sums = p1(start2d, end2d, full_edge_attr, table, batch_pad)

    p2 = pl.kernel(
        _pass2_body,
        out_type=[jax.ShapeDtypeStruct((NW * G,), jnp.float32)],
        mesh=mesh,
        compiler_params=pltpu.CompilerParams(needs_layout_passes=False,
                                             use_tc_tiling_on_sc=False),
        scratch_types=[
            pltpu.VMEM((NW * 640,), jnp.float32),
            pltpu.VMEM((G,), jnp.float32),
        ] + [pltpu.VMEM((CH,), jnp.float32)] * 6
          + [
            pltpu.VMEM((_L16 * G,), jnp.float32),
            pltpu.VMEM((G,), jnp.float32),
            pltpu.SemaphoreType.DMA,
        ],
    )
    stress, = p2(rec, sums)

    out = pl.pallas_call(
        _fin_body,
        out_shape=jax.ShapeDtypeStruct((1, 1), jnp.float32),
    )(sums.reshape(NW, 640), stress.reshape(NW, G))
    return out[0, 0]


# speedup vs baseline: 19.7361x; 1.0087x over previous
import functools

import jax
import jax.numpy as jnp
from jax import lax
from jax.experimental import pallas as pl
from jax.experimental.pallas import tpu as pltpu
from jax.experimental.pallas import tpu_sc as plsc

N = 100000
E = 6400000
G = 128
NW = 32
CH = 2048
SUB = 16
NCHUNK = E // CH
NB_SL = 3200
NNODE_IT = NB_SL // 16

_L16 = 16


def _iota16():
    return lax.iota(jnp.int32, _L16)


def _rsqrt(x):
    bits = plsc.bitcast(x, jnp.int32)
    y = plsc.bitcast(jnp.full((_L16,), 0x5F3759DF, jnp.int32) - jnp.right_shift(bits, 1), jnp.float32)
    half = x * 0.5
    for _ in range(3):
        y = y * (1.5 - half * y * y)
    return y


def _recip(x):
    bits = plsc.bitcast(x, jnp.int32)
    y = plsc.bitcast(jnp.full((_L16,), 0x7EF311C3, jnp.int32) - bits, jnp.float32)
    for _ in range(3):
        y = y * (2.0 - x * y)
    return y


def _wid():
    return lax.axis_index("s") * 2 + lax.axis_index("c")


def _nchunks(wid):
    return (NCHUNK - 1 - wid) // NW + 1


def _pass1_body(ei_hbm, attr_hbm, table_hbm, batch_hbm,
                rec_hbm, sums_hbm,
                s_idx, e_idx, attrb, rows_s, rows_e,
                r0, r1, r2, r3, r4, r5,
                acc_num, acc_den, acc_ov, acc_cnt, acc_gsz,
                nbuf, stage, sem):
    wid = _wid()
    lane = _iota16()
    lane128 = lane * G
    zeros16 = jnp.zeros((_L16,), jnp.float32)
    ones16 = jnp.ones((_L16,), jnp.float32)
    c0 = jnp.zeros((_L16,), jnp.int32)
    c1 = jnp.full((_L16,), 1, jnp.int32)
    c2 = jnp.full((_L16,), 2, jnp.int32)
    c3 = jnp.full((_L16,), 3, jnp.int32)
    c4 = jnp.full((_L16,), 4, jnp.int32)

    @pl.loop(0, G)
    def _zero(i):
        sl = pl.ds(i * _L16, _L16)
        acc_num[sl] = zeros16
        acc_den[sl] = zeros16
        acc_ov[sl] = zeros16
        acc_cnt[sl] = zeros16
        acc_gsz[sl] = zeros16

    nbase = jnp.minimum(wid * NB_SL, N - NB_SL)
    pltpu.sync_copy(batch_hbm.at[pl.ds(nbase, NB_SL)], nbuf)

    @pl.loop(0, NNODE_IT)
    def _hist(t):
        v = nbuf[pl.ds(t * _L16, _L16)]
        gid = nbase + t * _L16 + lane
        mask = jnp.logical_and(gid >= wid * NB_SL, gid < N)
        plsc.addupdate_scatter(acc_gsz, [lane128 + v], ones16, mask=mask)

    @pl.loop(0, _nchunks(wid))
    def _chunk(j):
        chunk = wid + NW * j
        off = chunk * CH
        icps = [pltpu.async_copy(attr_hbm.at[pl.ds(off, CH)], attrb, sem)]
        for k in range(SUB):
            icps.append(pltpu.async_copy(
                ei_hbm.at[0, pl.ds(off + k * 128, 128)], s_idx.at[k], sem))
            icps.append(pltpu.async_copy(
                ei_hbm.at[1, pl.ds(off + k * 128, 128)], e_idx.at[k], sem))
        for cp in icps:
            cp.wait()
        cps = []
        for k in range(SUB):
            cps.append(pltpu.async_copy(table_hbm.at[s_idx.at[k]], rows_s.at[k], sem))
            cps.append(pltpu.async_copy(table_hbm.at[e_idx.at[k]], rows_e.at[k], sem))
        for cp in cps:
            cp.wait()

        @pl.loop(0, CH // _L16)
        def _grp(g):
            base = g * _L16
            ev = base + lane
            jv = jnp.right_shift(ev, 7)
            rv = jnp.bitwise_and(ev, 127)
            px_s = plsc.load_gather(rows_s, [jv, rv, c0])
            py_s = plsc.load_gather(rows_s, [jv, rv, c1])
            sx_s = plsc.load_gather(rows_s, [jv, rv, c2])
            sy_s = plsc.load_gather(rows_s, [jv, rv, c3])
            b_s = plsc.bitcast(plsc.load_gather(rows_s, [jv, rv, c4]), jnp.int32)
            px_e = plsc.load_gather(rows_e, [jv, rv, c0])
            py_e = plsc.load_gather(rows_e, [jv, rv, c1])
            sx_e = plsc.load_gather(rows_e, [jv, rv, c2])
            sy_e = plsc.load_gather(rows_e, [jv, rv, c3])
            b_e = plsc.bitcast(plsc.load_gather(rows_e, [jv, rv, c4]), jnp.int32)
            d = plsc.load_gather(attrb, [ev, c0])

            invd = _recip(d)
            dx = px_s - px_e
            dy = py_s - py_e
            q = dx * dx + dy * dy
            eu = q * _rsqrt(q)
            r = eu * invd
            binv = lane128 + b_s
            plsc.addupdate_scatter(acc_num, [binv], r * r)
            plsc.addupdate_scatter(acc_den, [binv], r)

            ox = jnp.maximum((sx_s + sx_e) * 0.5 - jnp.abs(dx), 0.0)
            oy = jnp.maximum((sy_s + sy_e) * 0.5 - jnp.abs(dy), 0.0)
            tot = sx_s + sy_s + sx_e + sy_e
            nov = ox * oy * _recip(tot)
            plsc.addupdate_scatter(acc_ov, [binv], nov)
            plsc.addupdate_scatter(acc_cnt, [binv], ones16)

            sl = pl.ds(base, _L16)
            r0[sl] = px_s
            r1[sl] = py_s
            r2[sl] = px_e
            r3[sl] = py_e
            r4[sl] = plsc.bitcast(jnp.left_shift(b_s, 7) + b_e, jnp.float32)
            r5[sl] = invd

        for q_i, buf in enumerate((r0, r1, r2, r3, r4, r5)):
            pltpu.sync_copy(buf, rec_hbm.at[q_i, pl.ds(off, CH)])

    for q_i, acc in enumerate((acc_num, acc_den, acc_ov, acc_cnt, acc_gsz)):
        for blk in range(G // _L16):
            tot = acc[pl.ds(blk * _L16, _L16)]
            for l in range(1, _L16):
                tot = tot + acc[pl.ds(l * G + blk * _L16, _L16)]
            stage[pl.ds(q_i * G + blk * _L16, _L16)] = tot
    pltpu.sync_copy(stage, sums_hbm.at[pl.ds(wid * 640, 640)])


def _pass2_body(rec_hbm, sums_hbm, stress_hbm,
                sbuf, invs, b0, b1, b2, b3, b4, b5, acc_st, stage, sem):
    wid = _wid()
    lane = _iota16()
    lane128 = lane * G
    zeros16 = jnp.zeros((_L16,), jnp.float32)

    @pl.loop(0, G)
    def _zero(i):
        acc_st[pl.ds(i * _L16, _L16)] = zeros16

    pltpu.sync_copy(sums_hbm, sbuf)
    for blk in range(G // _L16):
        ntot = sbuf[pl.ds(blk * _L16, _L16)]
        dtot = sbuf[pl.ds(G + blk * _L16, _L16)]
        for w in range(1, NW):
            ntot = ntot + sbuf[pl.ds(w * 640 + blk * _L16, _L16)]
            dtot = dtot + sbuf[pl.ds(w * 640 + G + blk * _L16, _L16)]
        invs[pl.ds(blk * _L16, _L16)] = dtot * _recip(ntot)

    @pl.loop(0, _nchunks(wid))
    def _chunk(j):
        chunk = wid + NW * j
        off = chunk * CH
        bufs = (b0, b1, b2, b3, b4, b5)
        cps = [pltpu.async_copy(rec_hbm.at[q_i, pl.ds(off, CH)], bufs[q_i], sem)
               for q_i in range(6)]
        for cp in cps:
            cp.wait()

        @pl.loop(0, CH // _L16)
        def _grp(g):
            sl = pl.ds(g * _L16, _L16)
            px_s = b0[sl]
            py_s = b1[sl]
            px_e = b2[sl]
            py_e = b3[sl]
            bp = plsc.bitcast(b4[sl], jnp.int32)
            invd = b5[sl]
            b_s = jnp.right_shift(bp, 7)
            b_e = jnp.bitwise_and(bp, 127)
            u = plsc.load_gather(invs, [b_s])
            v = plsc.load_gather(invs, [b_e])
            ddx = px_s * u - px_e * v
            ddy = py_s * u - py_e * v
            q2 = ddx * ddx + ddy * ddy
            eu2 = q2 * _rsqrt(q2)
            t = eu2 * invd - 1.0
            plsc.addupdate_scatter(acc_st, [lane128 + b_s], t * t)

    for blk in range(G // _L16):
        tot = acc_st[pl.ds(blk * _L16, _L16)]
        for l in range(1, _L16):
            tot = tot + acc_st[pl.ds(l * G + blk * _L16, _L16)]
        stage[pl.ds(blk * _L16, _L16)] = tot
    pltpu.sync_copy(stage, stress_hbm.at[pl.ds(wid * G, G)])


def _fin_body(sums_ref, stress_ref, o_ref):
    s = sums_ref[:]
    ov = jnp.sum(s[:, 2 * G:3 * G], axis=0)
    cnt = jnp.sum(s[:, 3 * G:4 * G], axis=0)
    gsz = jnp.sum(s[:, 4 * G:5 * G], axis=0)
    st = jnp.sum(stress_ref[:], axis=0)
    combined = st / (gsz * gsz) + ov / cnt
    o_ref[:, :] = jnp.mean(combined)[None, None]


@jax.jit
def kernel(node_pos, node_sizes, full_edge_index, batch, full_edge_attr):
    table = jnp.concatenate(
        [node_pos, node_sizes,
         lax.bitcast_convert_type(batch, jnp.float32)[:, None],
         jnp.zeros((N, 11), jnp.float32)], axis=1)
    mesh = plsc.VectorSubcoreMesh(core_axis_name="c", subcore_axis_name="s")

    p1 = pl.kernel(
        _pass1_body,
        out_type=[jax.ShapeDtypeStruct((6, E), jnp.float32),
                  jax.ShapeDtypeStruct((NW * 640,), jnp.float32)],
        mesh=mesh,
        compiler_params=pltpu.CompilerParams(needs_layout_passes=False,
                                             use_tc_tiling_on_sc=False),
        scratch_types=[
            pltpu.VMEM((SUB, 128), jnp.int32),
            pltpu.VMEM((SUB, 128), jnp.int32),
            pltpu.VMEM((CH, 4), jnp.float32),
            pltpu.VMEM((SUB, 128, 16), jnp.float32),
            pltpu.VMEM((SUB, 128, 16), jnp.float32),
        ] + [pltpu.VMEM((CH,), jnp.float32)] * 6
          + [pltpu.VMEM((_L16 * G,), jnp.float32)] * 5
          + [
            pltpu.VMEM((NB_SL,), jnp.int32),
            pltpu.VMEM((640,), jnp.float32),
            pltpu.SemaphoreType.DMA,
        ],
    )
    rec, sums = p1(full_edge_index, full_edge_attr, table, batch)

    p2 = pl.kernel(
        _pass2_body,
        out_type=[jax.ShapeDtypeStruct((NW * G,), jnp.float32)],
        mesh=mesh,
        compiler_params=pltpu.CompilerParams(needs_layout_passes=False,
                                             use_tc_tiling_on_sc=False),
        scratch_types=[
            pltpu.VMEM((NW * 640,), jnp.float32),
            pltpu.VMEM((G,), jnp.float32),
        ] + [pltpu.VMEM((CH,), jnp.float32)] * 6
          + [
            pltpu.VMEM((_L16 * G,), jnp.float32),
            pltpu.VMEM((G,), jnp.float32),
            pltpu.SemaphoreType.DMA,
        ],
    )
    stress, = p2(rec, sums)

    out = pl.pallas_call(
        _fin_body,
        out_shape=jax.ShapeDtypeStruct((1, 1), jnp.float32),
    )(sums.reshape(NW, 640), stress.reshape(NW, G))
    return out[0, 0]


# speedup vs baseline: 102.4074x; 5.1888x over previous
import functools

import jax
import jax.numpy as jnp
from jax import lax
from jax.experimental import pallas as pl
from jax.experimental.pallas import tpu as pltpu
from jax.experimental.pallas import tpu_sc as plsc

N = 100000
E = 6400000
G = 128
NW = 32
CH = 1024
SUB = CH // 128
NCHUNK = E // CH
TRIPS = -(-NCHUNK // NW) * NW // NW
PAIRS = TRIPS // 2
NB_SL = 3200
NNODE_IT = NB_SL // 16
NGRP = CH // 16

_L16 = 16


def _iota16():
    return lax.iota(jnp.int32, _L16)


def _rsqrt(x):
    bits = plsc.bitcast(x, jnp.int32)
    y = plsc.bitcast(jnp.full((_L16,), 0x5F3759DF, jnp.int32) - jnp.right_shift(bits, 1), jnp.float32)
    half = x * 0.5
    for _ in range(3):
        y = y * (1.5 - half * y * y)
    return y


def _recip(x):
    bits = plsc.bitcast(x, jnp.int32)
    y = plsc.bitcast(jnp.full((_L16,), 0x7EF311C3, jnp.int32) - bits, jnp.float32)
    for _ in range(3):
        y = y * (2.0 - x * y)
    return y


def _wid():
    return lax.axis_index("s") * 2 + lax.axis_index("c")


def _pass1_body(start_hbm, end_hbm, d_hbm, table_hbm, batch_hbm,
                rec_hbm, sums_hbm,
                sxa, exa, sxb, exb, dba, dbb,
                rsa, rea, rsb, reb,
                reca, recb,
                acc_num, acc_den, acc_ov, acc_cnt, acc_gsz,
                nbuf, stage, sem_in, sem_g, sem_o):
    wid = _wid()
    lane = _iota16()
    lane128 = lane * G
    zeros16 = jnp.zeros((_L16,), jnp.float32)
    ones16 = jnp.ones((_L16,), jnp.float32)
    c0 = jnp.zeros((_L16,), jnp.int32)
    c1 = jnp.full((_L16,), 1, jnp.int32)
    c2 = jnp.full((_L16,), 2, jnp.int32)
    c3 = jnp.full((_L16,), 3, jnp.int32)
    c4 = jnp.full((_L16,), 4, jnp.int32)

    @pl.loop(0, G)
    def _zero(i):
        sl = pl.ds(i * _L16, _L16)
        acc_num[sl] = zeros16
        acc_den[sl] = zeros16
        acc_ov[sl] = zeros16
        acc_cnt[sl] = zeros16
        acc_gsz[sl] = zeros16

    nbase = jnp.minimum(wid * NB_SL, N - NB_SL)
    pltpu.sync_copy(batch_hbm.at[pl.ds(nbase, NB_SL)], nbuf)

    @pl.loop(0, NNODE_IT)
    def _hist(t):
        v = nbuf[pl.ds(t * _L16, _L16)]
        gid = nbase + t * _L16 + lane
        mask = jnp.logical_and(gid >= wid * NB_SL, gid < N)
        plsc.addupdate_scatter(acc_gsz, [lane128 + v], ones16, mask=mask)

    def chunk_of(t):
        cid_raw = wid + NW * t
        valid = cid_raw < NCHUNK
        return jnp.where(valid, cid_raw, wid), valid

    def fire_in(cid, sx, ex, db):
        off = cid * CH
        pltpu.async_copy(d_hbm.at[pl.ds(off, CH)], db, sem_in)
        for k in range(SUB):
            pltpu.async_copy(start_hbm.at[pl.ds(off + k * 128, 128)], sx.at[k], sem_in)
            pltpu.async_copy(end_hbm.at[pl.ds(off + k * 128, 128)], ex.at[k], sem_in)

    def wait_in(sx, ex, db):
        pltpu.make_async_copy(d_hbm.at[pl.ds(0, CH)], db, sem_in).wait()
        for k in range(SUB):
            pltpu.make_async_copy(start_hbm.at[pl.ds(0, 128)], sx.at[k], sem_in).wait()
            pltpu.make_async_copy(end_hbm.at[pl.ds(0, 128)], ex.at[k], sem_in).wait()

    def fire_gather(sx, ex, rs, re):
        for k in range(SUB):
            pltpu.async_copy(table_hbm.at[sx.at[k]], rs.at[k], sem_g)
            pltpu.async_copy(table_hbm.at[ex.at[k]], re.at[k], sem_g)

    def wait_gather(rs, re):
        for k in range(SUB):
            pltpu.make_async_copy(table_hbm.at[pl.ds(0, 128)], rs.at[k], sem_g).wait()
            pltpu.make_async_copy(table_hbm.at[pl.ds(0, 128)], re.at[k], sem_g).wait()

    def fire_out(cid, bufs):
        off = cid * CH
        for q_i, b in enumerate(bufs):
            pltpu.async_copy(b, rec_hbm.at[q_i, pl.ds(off, CH)], sem_o)

    def wait_out(bufs):
        for q_i, b in enumerate(bufs):
            pltpu.make_async_copy(b, rec_hbm.at[q_i, pl.ds(0, CH)], sem_o).wait()

    def compute(valid, db, rs, re, bufs):
        limit = jnp.where(valid, _L16, 0)
        vmask = lane < limit
        r0, r1, r2, r3, r4, r5 = bufs

        @pl.loop(0, NGRP)
        def _grp(g):
            base = g * _L16
            ev = base + lane
            jv = jnp.right_shift(ev, 7)
            rv = jnp.bitwise_and(ev, 127)
            px_s = plsc.load_gather(rs, [jv, rv, c0])
            py_s = plsc.load_gather(rs, [jv, rv, c1])
            sx_s = plsc.load_gather(rs, [jv, rv, c2])
            sy_s = plsc.load_gather(rs, [jv, rv, c3])
            b_s = plsc.bitcast(plsc.load_gather(rs, [jv, rv, c4]), jnp.int32)
            px_e = plsc.load_gather(re, [jv, rv, c0])
            py_e = plsc.load_gather(re, [jv, rv, c1])
            sx_e = plsc.load_gather(re, [jv, rv, c2])
            sy_e = plsc.load_gather(re, [jv, rv, c3])
            b_e = plsc.bitcast(plsc.load_gather(re, [jv, rv, c4]), jnp.int32)
            d = db[pl.ds(base, _L16)]

            invd = _recip(d)
            dx = px_s - px_e
            dy = py_s - py_e
            q = dx * dx + dy * dy
            eu = q * _rsqrt(q)
            r = eu * invd
            binv = lane128 + b_s
            plsc.addupdate_scatter(acc_num, [binv], r * r, mask=vmask)
            plsc.addupdate_scatter(acc_den, [binv], r, mask=vmask)

            ox = jnp.maximum((sx_s + sx_e) * 0.5 - jnp.abs(dx), 0.0)
            oy = jnp.maximum((sy_s + sy_e) * 0.5 - jnp.abs(dy), 0.0)
            tot = sx_s + sy_s + sx_e + sy_e
            nov = ox * oy * _recip(tot)
            plsc.addupdate_scatter(acc_ov, [binv], nov, mask=vmask)
            plsc.addupdate_scatter(acc_cnt, [binv], ones16, mask=vmask)

            sl = pl.ds(base, _L16)
            r0[sl] = px_s
            r1[sl] = py_s
            r2[sl] = px_e
            r3[sl] = py_e
            r4[sl] = plsc.bitcast(jnp.left_shift(b_s, 7) + b_e, jnp.float32)
            r5[sl] = invd

    bufsA = (reca.at[0], reca.at[1], reca.at[2], reca.at[3], reca.at[4], reca.at[5])
    bufsB = (recb.at[0], recb.at[1], recb.at[2], recb.at[3], recb.at[4], recb.at[5])

    cid0, _ = chunk_of(0)
    fire_in(cid0, sxa, exa, dba)

    @pl.loop(0, PAIRS)
    def _pair(p):
        cidA, _va = chunk_of(2 * p)
        cidB, vB = chunk_of(2 * p + 1)
        wait_in(sxa, exa, dba)
        fire_gather(sxa, exa, rsa, rea)
        fire_in(cidB, sxb, exb, dbb)
        wait_gather(rsa, rea)

        @pl.when(p > 0)
        def _():
            wait_out(bufsA)
        compute(True, dba, rsa, rea, bufsA)
        fire_out(cidA, bufsA)
        wait_in(sxb, exb, dbb)
        fire_gather(sxb, exb, rsb, reb)

        @pl.when(p < PAIRS - 1)
        def _():
            cidA2, _ = chunk_of(2 * p + 2)
            fire_in(cidA2, sxa, exa, dba)
        wait_gather(rsb, reb)

        @pl.when(p > 0)
        def _():
            wait_out(bufsB)
        compute(vB, dbb, rsb, reb, bufsB)
        fire_out(cidB, bufsB)

    wait_out(bufsA)
    wait_out(bufsB)

    for q_i, acc in enumerate((acc_num, acc_den, acc_ov, acc_cnt, acc_gsz)):
        for blk in range(G // _L16):
            tot = acc[pl.ds(blk * _L16, _L16)]
            for l in range(1, _L16):
                tot = tot + acc[pl.ds(l * G + blk * _L16, _L16)]
            stage[pl.ds(q_i * G + blk * _L16, _L16)] = tot
    pltpu.sync_copy(stage, sums_hbm.at[pl.ds(wid * 640, 640)])


def _pass2_body(rec_hbm, sums_hbm, stress_hbm,
                sbuf, invs, reca, recb, acc_st, stage, sem_in):
    wid = _wid()
    lane = _iota16()
    lane128 = lane * G
    zeros16 = jnp.zeros((_L16,), jnp.float32)

    @pl.loop(0, G)
    def _zero(i):
        acc_st[pl.ds(i * _L16, _L16)] = zeros16

    pltpu.sync_copy(sums_hbm, sbuf)
    for blk in range(G // _L16):
        ntot = sbuf[pl.ds(blk * _L16, _L16)]
        dtot = sbuf[pl.ds(G + blk * _L16, _L16)]
        for w in range(1, NW):
            ntot = ntot + sbuf[pl.ds(w * 640 + blk * _L16, _L16)]
            dtot = dtot + sbuf[pl.ds(w * 640 + G + blk * _L16, _L16)]
        invs[pl.ds(blk * _L16, _L16)] = dtot * _recip(ntot)

    def chunk_of(t):
        cid_raw = wid + NW * t
        valid = cid_raw < NCHUNK
        return jnp.where(valid, cid_raw, wid), valid

    def fire_rin(cid, bufs):
        off = cid * CH
        for q_i in range(6):
            pltpu.async_copy(rec_hbm.at[q_i, pl.ds(off, CH)], bufs.at[q_i], sem_in)

    def wait_rin(bufs):
        for q_i in range(6):
            pltpu.make_async_copy(rec_hbm.at[0, pl.ds(0, CH)], bufs.at[q_i], sem_in).wait()

    def compute(valid, bufs):
        limit = jnp.where(valid, _L16, 0)
        vmask = lane < limit

        @pl.loop(0, NGRP)
        def _grp(g):
            sl = pl.ds(g * _L16, _L16)
            px_s = bufs[0, sl]
            py_s = bufs[1, sl]
            px_e = bufs[2, sl]
            py_e = bufs[3, sl]
            bp = plsc.bitcast(bufs[4, sl], jnp.int32)
            invd = bufs[5, sl]
            b_s = jnp.right_shift(bp, 7)
            b_e = jnp.bitwise_and(bp, 127)
            u = plsc.load_gather(invs, [b_s])
            v = plsc.load_gather(invs, [b_e])
            ddx = px_s * u - px_e * v
            ddy = py_s * u - py_e * v
            q2 = ddx * ddx + ddy * ddy
            eu2 = q2 * _rsqrt(q2)
            t = eu2 * invd - 1.0
            plsc.addupdate_scatter(acc_st, [lane128 + b_s], t * t, mask=vmask)

    cid0, _ = chunk_of(0)
    fire_rin(cid0, reca)

    @pl.loop(0, PAIRS)
    def _pair(p):
        _cidA, _vA = chunk_of(2 * p)
        cidB, vB = chunk_of(2 * p + 1)
        wait_rin(reca)
        fire_rin(cidB, recb)
        compute(True, reca)
        wait_rin(recb)

        @pl.when(p < PAIRS - 1)
        def _():
            cidA2, _ = chunk_of(2 * p + 2)
            fire_rin(cidA2, reca)
        compute(vB, recb)

    for blk in range(G // _L16):
        tot = acc_st[pl.ds(blk * _L16, _L16)]
        for l in range(1, _L16):
            tot = tot + acc_st[pl.ds(l * G + blk * _L16, _L16)]
        stage[pl.ds(blk * _L16, _L16)] = tot
    pltpu.sync_copy(stage, stress_hbm.at[pl.ds(wid * G, G)])


def _fin_body(sums_ref, stress_ref, o_ref):
    s = sums_ref[:]
    ov = jnp.sum(s[:, 2 * G:3 * G], axis=0)
    cnt = jnp.sum(s[:, 3 * G:4 * G], axis=0)
    gsz = jnp.sum(s[:, 4 * G:5 * G], axis=0)
    st = jnp.sum(stress_ref[:], axis=0)
    combined = st / (gsz * gsz) + ov / cnt
    o_ref[:, :] = jnp.mean(combined)[None, None]


@jax.jit
def kernel(node_pos, node_sizes, full_edge_index, batch, full_edge_attr):
    table = jnp.concatenate(
        [node_pos, node_sizes,
         lax.bitcast_convert_type(batch, jnp.float32)[:, None],
         jnp.zeros((N, 11), jnp.float32)], axis=1)

    mesh = plsc.VectorSubcoreMesh(core_axis_name="c", subcore_axis_name="s")

    p1 = pl.kernel(
        _pass1_body,
        out_type=[jax.ShapeDtypeStruct((6, E), jnp.float32),
                  jax.ShapeDtypeStruct((NW * 640,), jnp.float32)],
        mesh=mesh,
        compiler_params=pltpu.CompilerParams(needs_layout_passes=False,
                                             use_tc_tiling_on_sc=False),
        scratch_types=[
            pltpu.VMEM((SUB, 128), jnp.int32),
            pltpu.VMEM((SUB, 128), jnp.int32),
            pltpu.VMEM((SUB, 128), jnp.int32),
            pltpu.VMEM((SUB, 128), jnp.int32),
            pltpu.VMEM((CH,), jnp.float32),
            pltpu.VMEM((CH,), jnp.float32),
            pltpu.VMEM((SUB, 128, 16), jnp.float32),
            pltpu.VMEM((SUB, 128, 16), jnp.float32),
            pltpu.VMEM((SUB, 128, 16), jnp.float32),
            pltpu.VMEM((SUB, 128, 16), jnp.float32),
            pltpu.VMEM((6, CH), jnp.float32),
            pltpu.VMEM((6, CH), jnp.float32),
        ] + [pltpu.VMEM((_L16 * G,), jnp.float32)] * 5
          + [
            pltpu.VMEM((NB_SL,), jnp.int32),
            pltpu.VMEM((640,), jnp.float32),
            pltpu.SemaphoreType.DMA,
            pltpu.SemaphoreType.DMA,
            pltpu.SemaphoreType.DMA,
        ],
    )
    rec, sums = p1(full_edge_index[0], full_edge_index[1],
                   full_edge_attr[:, 0], table, batch)

    p2 = pl.kernel(
        _pass2_body,
        out_type=[jax.ShapeDtypeStruct((NW * G,), jnp.float32)],
        mesh=mesh,
        compiler_params=pltpu.CompilerParams(needs_layout_passes=False,
                                             use_tc_tiling_on_sc=False),
        scratch_types=[
            pltpu.VMEM((NW * 640,), jnp.float32),
            pltpu.VMEM((G,), jnp.float32),
            pltpu.VMEM((6, CH), jnp.float32),
            pltpu.VMEM((6, CH), jnp.float32),
            pltpu.VMEM((_L16 * G,), jnp.float32),
            pltpu.VMEM((G,), jnp.float32),
            pltpu.SemaphoreType.DMA,
        ],
    )
    stress, = p2(rec, sums)

    out = pl.pallas_call(
        _fin_body,
        out_shape=jax.ShapeDtypeStruct((1, 1), jnp.float32),
    )(sums.reshape(NW, 640), stress.reshape(NW, G))
    return out[0, 0]


# speedup vs baseline: 102.5247x; 1.0011x over previous
import functools

import jax
import jax.numpy as jnp
from jax import lax
from jax.experimental import pallas as pl
from jax.experimental.pallas import tpu as pltpu
from jax.experimental.pallas import tpu_sc as plsc

N = 100000
E = 6400000
G = 128
NW = 32
CH = 1024
SUB = CH // 128
NCHUNK = E // CH
TRIPS = -(-NCHUNK // NW) * NW // NW
PAIRS = TRIPS // 2
CH2 = 5120
NCHUNK2 = E // CH2
TRIPS2 = -(-NCHUNK2 // NW)
PAIRS2 = TRIPS2 // 2
NGRP2 = CH2 // 16
NB_SL = 3200
NNODE_IT = NB_SL // 16
NGRP = CH // 16

_L16 = 16


def _iota16():
    return lax.iota(jnp.int32, _L16)


def _rsqrt(x):
    bits = plsc.bitcast(x, jnp.int32)
    y = plsc.bitcast(jnp.full((_L16,), 0x5F3759DF, jnp.int32) - jnp.right_shift(bits, 1), jnp.float32)
    half = x * 0.5
    for _ in range(3):
        y = y * (1.5 - half * y * y)
    return y


def _recip(x):
    bits = plsc.bitcast(x, jnp.int32)
    y = plsc.bitcast(jnp.full((_L16,), 0x7EF311C3, jnp.int32) - bits, jnp.float32)
    for _ in range(3):
        y = y * (2.0 - x * y)
    return y


def _wid():
    return lax.axis_index("s") * 2 + lax.axis_index("c")


def _pass1_body(start_hbm, end_hbm, d_hbm, table_hbm, batch_hbm,
                rec_hbm, sums_hbm,
                sxa, exa, sxb, exb, dba, dbb,
                rsa, rea, rsb, reb,
                reca, recb,
                acc_num, acc_den, acc_ov, acc_cnt, acc_gsz,
                nbuf, stage, sem_in, sem_g, sem_o):
    wid = _wid()
    lane = _iota16()
    lane128 = lane * G
    zeros16 = jnp.zeros((_L16,), jnp.float32)
    ones16 = jnp.ones((_L16,), jnp.float32)
    c0 = jnp.zeros((_L16,), jnp.int32)
    c1 = jnp.full((_L16,), 1, jnp.int32)
    c2 = jnp.full((_L16,), 2, jnp.int32)
    c3 = jnp.full((_L16,), 3, jnp.int32)
    c4 = jnp.full((_L16,), 4, jnp.int32)

    @pl.loop(0, G)
    def _zero(i):
        sl = pl.ds(i * _L16, _L16)
        acc_num[sl] = zeros16
        acc_den[sl] = zeros16
        acc_ov[sl] = zeros16
        acc_cnt[sl] = zeros16
        acc_gsz[sl] = zeros16

    nbase = jnp.minimum(wid * NB_SL, N - NB_SL)
    pltpu.sync_copy(batch_hbm.at[pl.ds(nbase, NB_SL)], nbuf)

    @pl.loop(0, NNODE_IT)
    def _hist(t):
        v = nbuf[pl.ds(t * _L16, _L16)]
        gid = nbase + t * _L16 + lane
        mask = jnp.logical_and(gid >= wid * NB_SL, gid < N)
        plsc.addupdate_scatter(acc_gsz, [lane128 + v], ones16, mask=mask)

    def chunk_of(t):
        cid_raw = wid + NW * t
        valid = cid_raw < NCHUNK
        return jnp.where(valid, cid_raw, wid), valid

    def fire_in(cid, sx, ex, db):
        off = cid * CH
        pltpu.async_copy(d_hbm.at[pl.ds(off, CH)], db, sem_in)
        for k in range(SUB):
            pltpu.async_copy(start_hbm.at[pl.ds(off + k * 128, 128)], sx.at[k], sem_in)
            pltpu.async_copy(end_hbm.at[pl.ds(off + k * 128, 128)], ex.at[k], sem_in)

    def wait_in(sx, ex, db):
        pltpu.make_async_copy(d_hbm.at[pl.ds(0, CH)], db, sem_in).wait()
        for k in range(SUB):
            pltpu.make_async_copy(start_hbm.at[pl.ds(0, 128)], sx.at[k], sem_in).wait()
            pltpu.make_async_copy(end_hbm.at[pl.ds(0, 128)], ex.at[k], sem_in).wait()

    def fire_gather(sx, ex, rs, re):
        for k in range(SUB):
            pltpu.async_copy(table_hbm.at[sx.at[k]], rs.at[k], sem_g)
            pltpu.async_copy(table_hbm.at[ex.at[k]], re.at[k], sem_g)

    def wait_gather(rs, re):
        for k in range(SUB):
            pltpu.make_async_copy(table_hbm.at[pl.ds(0, 128)], rs.at[k], sem_g).wait()
            pltpu.make_async_copy(table_hbm.at[pl.ds(0, 128)], re.at[k], sem_g).wait()

    def fire_out(cid, bufs):
        off = cid * CH
        for q_i, b in enumerate(bufs):
            pltpu.async_copy(b, rec_hbm.at[q_i, pl.ds(off, CH)], sem_o)

    def wait_out(bufs):
        for q_i, b in enumerate(bufs):
            pltpu.make_async_copy(b, rec_hbm.at[q_i, pl.ds(0, CH)], sem_o).wait()

    def compute(valid, db, rs, re, bufs):
        limit = jnp.where(valid, _L16, 0)
        vmask = lane < limit
        r0, r1, r2, r3, r4, r5 = bufs

        @pl.loop(0, NGRP)
        def _grp(g):
            base = g * _L16
            ev = base + lane
            jv = jnp.right_shift(ev, 7)
            rv = jnp.bitwise_and(ev, 127)
            px_s = plsc.load_gather(rs, [jv, rv, c0])
            py_s = plsc.load_gather(rs, [jv, rv, c1])
            sx_s = plsc.load_gather(rs, [jv, rv, c2])
            sy_s = plsc.load_gather(rs, [jv, rv, c3])
            b_s = plsc.bitcast(plsc.load_gather(rs, [jv, rv, c4]), jnp.int32)
            px_e = plsc.load_gather(re, [jv, rv, c0])
            py_e = plsc.load_gather(re, [jv, rv, c1])
            sx_e = plsc.load_gather(re, [jv, rv, c2])
            sy_e = plsc.load_gather(re, [jv, rv, c3])
            b_e = plsc.bitcast(plsc.load_gather(re, [jv, rv, c4]), jnp.int32)
            d = db[pl.ds(base, _L16)]

            invd = _recip(d)
            dx = px_s - px_e
            dy = py_s - py_e
            q = dx * dx + dy * dy
            eu = q * _rsqrt(q)
            r = eu * invd
            binv = lane128 + b_s
            plsc.addupdate_scatter(acc_num, [binv], r * r, mask=vmask)
            plsc.addupdate_scatter(acc_den, [binv], r, mask=vmask)

            ox = jnp.maximum((sx_s + sx_e) * 0.5 - jnp.abs(dx), 0.0)
            oy = jnp.maximum((sy_s + sy_e) * 0.5 - jnp.abs(dy), 0.0)
            tot = sx_s + sy_s + sx_e + sy_e
            nov = ox * oy * _recip(tot)
            plsc.addupdate_scatter(acc_ov, [binv], nov, mask=vmask)
            plsc.addupdate_scatter(acc_cnt, [binv], ones16, mask=vmask)

            sl = pl.ds(base, _L16)
            r0[sl] = px_s
            r1[sl] = py_s
            r2[sl] = px_e
            r3[sl] = py_e
            r4[sl] = plsc.bitcast(jnp.left_shift(b_s, 7) + b_e, jnp.float32)
            r5[sl] = invd

    bufsA = (reca.at[0], reca.at[1], reca.at[2], reca.at[3], reca.at[4], reca.at[5])
    bufsB = (recb.at[0], recb.at[1], recb.at[2], recb.at[3], recb.at[4], recb.at[5])

    cid0, _ = chunk_of(0)
    fire_in(cid0, sxa, exa, dba)

    @pl.loop(0, PAIRS)
    def _pair(p):
        cidA, _va = chunk_of(2 * p)
        cidB, vB = chunk_of(2 * p + 1)
        wait_in(sxa, exa, dba)
        fire_gather(sxa, exa, rsa, rea)
        fire_in(cidB, sxb, exb, dbb)
        wait_gather(rsa, rea)

        @pl.when(p > 0)
        def _():
            wait_out(bufsA)
        compute(True, dba, rsa, rea, bufsA)
        fire_out(cidA, bufsA)
        wait_in(sxb, exb, dbb)
        fire_gather(sxb, exb, rsb, reb)

        @pl.when(p < PAIRS - 1)
        def _():
            cidA2, _ = chunk_of(2 * p + 2)
            fire_in(cidA2, sxa, exa, dba)
        wait_gather(rsb, reb)

        @pl.when(p > 0)
        def _():
            wait_out(bufsB)
        compute(vB, dbb, rsb, reb, bufsB)
        fire_out(cidB, bufsB)

    wait_out(bufsA)
    wait_out(bufsB)

    for q_i, acc in enumerate((acc_num, acc_den, acc_ov, acc_cnt, acc_gsz)):
        for blk in range(G // _L16):
            tot = acc[pl.ds(blk * _L16, _L16)]
            for l in range(1, _L16):
                tot = tot + acc[pl.ds(l * G + blk * _L16, _L16)]
            stage[pl.ds(q_i * G + blk * _L16, _L16)] = tot
    pltpu.sync_copy(stage, sums_hbm.at[pl.ds(wid * 640, 640)])


def _pass2_body(rec_hbm, sums_hbm, stress_hbm,
                sbuf, invs, reca, recb, acc_st, stage, sem_in):
    wid = _wid()
    lane = _iota16()
    lane128 = lane * G
    zeros16 = jnp.zeros((_L16,), jnp.float32)

    @pl.loop(0, G)
    def _zero(i):
        acc_st[pl.ds(i * _L16, _L16)] = zeros16

    pltpu.sync_copy(sums_hbm, sbuf)
    for blk in range(G // _L16):
        ntot = sbuf[pl.ds(blk * _L16, _L16)]
        dtot = sbuf[pl.ds(G + blk * _L16, _L16)]
        for w in range(1, NW):
            ntot = ntot + sbuf[pl.ds(w * 640 + blk * _L16, _L16)]
            dtot = dtot + sbuf[pl.ds(w * 640 + G + blk * _L16, _L16)]
        invs[pl.ds(blk * _L16, _L16)] = dtot * _recip(ntot)

    def chunk_of(t):
        cid_raw = wid + NW * t
        valid = cid_raw < NCHUNK2
        return jnp.where(valid, cid_raw, wid), valid

    def fire_rin(cid, bufs):
        off = cid * CH2
        for q_i in range(6):
            pltpu.async_copy(rec_hbm.at[q_i, pl.ds(off, CH2)], bufs.at[q_i], sem_in)

    def wait_rin(bufs):
        for q_i in range(6):
            pltpu.make_async_copy(rec_hbm.at[0, pl.ds(0, CH2)], bufs.at[q_i], sem_in).wait()

    def compute(valid, bufs):
        limit = jnp.where(valid, _L16, 0)
        vmask = lane < limit

        @pl.loop(0, NGRP2, unroll=4)
        def _grp(g):
            sl = pl.ds(g * _L16, _L16)
            px_s = bufs[0, sl]
            py_s = bufs[1, sl]
            px_e = bufs[2, sl]
            py_e = bufs[3, sl]
            bp = plsc.bitcast(bufs[4, sl], jnp.int32)
            invd = bufs[5, sl]
            b_s = jnp.right_shift(bp, 7)
            b_e = jnp.bitwise_and(bp, 127)
            u = plsc.load_gather(invs, [b_s])
            v = plsc.load_gather(invs, [b_e])
            ddx = px_s * u - px_e * v
            ddy = py_s * u - py_e * v
            q2 = ddx * ddx + ddy * ddy
            eu2 = q2 * _rsqrt(q2)
            t = eu2 * invd - 1.0
            plsc.addupdate_scatter(acc_st, [lane128 + b_s], t * t, mask=vmask)

    cid0, _ = chunk_of(0)
    fire_rin(cid0, reca)

    @pl.loop(0, PAIRS2)
    def _pair(p):
        _cidA, vA = chunk_of(2 * p)
        cidB, vB = chunk_of(2 * p + 1)
        wait_rin(reca)
        fire_rin(cidB, recb)
        compute(vA, reca)
        wait_rin(recb)

        @pl.when(p < PAIRS2 - 1)
        def _():
            cidA2, _ = chunk_of(2 * p + 2)
            fire_rin(cidA2, reca)
        compute(vB, recb)

    for blk in range(G // _L16):
        tot = acc_st[pl.ds(blk * _L16, _L16)]
        for l in range(1, _L16):
            tot = tot + acc_st[pl.ds(l * G + blk * _L16, _L16)]
        stage[pl.ds(blk * _L16, _L16)] = tot
    pltpu.sync_copy(stage, stress_hbm.at[pl.ds(wid * G, G)])


def _fin_body(sums_ref, stress_ref, o_ref):
    s = sums_ref[:]
    ov = jnp.sum(s[:, 2 * G:3 * G], axis=0)
    cnt = jnp.sum(s[:, 3 * G:4 * G], axis=0)
    gsz = jnp.sum(s[:, 4 * G:5 * G], axis=0)
    st = jnp.sum(stress_ref[:], axis=0)
    combined = st / (gsz * gsz) + ov / cnt
    o_ref[:, :] = jnp.mean(combined)[None, None]


@jax.jit
def kernel(node_pos, node_sizes, full_edge_index, batch, full_edge_attr):
    table = jnp.concatenate(
        [node_pos, node_sizes,
         lax.bitcast_convert_type(batch, jnp.float32)[:, None],
         jnp.zeros((N, 11), jnp.float32)], axis=1)

    mesh = plsc.VectorSubcoreMesh(core_axis_name="c", subcore_axis_name="s")

    p1 = pl.kernel(
        _pass1_body,
        out_type=[jax.ShapeDtypeStruct((6, E), jnp.float32),
                  jax.ShapeDtypeStruct((NW * 640,), jnp.float32)],
        mesh=mesh,
        compiler_params=pltpu.CompilerParams(needs_layout_passes=False,
                                             use_tc_tiling_on_sc=False),
        scratch_types=[
            pltpu.VMEM((SUB, 128), jnp.int32),
            pltpu.VMEM((SUB, 128), jnp.int32),
            pltpu.VMEM((SUB, 128), jnp.int32),
            pltpu.VMEM((SUB, 128), jnp.int32),
            pltpu.VMEM((CH,), jnp.float32),
            pltpu.VMEM((CH,), jnp.float32),
            pltpu.VMEM((SUB, 128, 16), jnp.float32),
            pltpu.VMEM((SUB, 128, 16), jnp.float32),
            pltpu.VMEM((SUB, 128, 16), jnp.float32),
            pltpu.VMEM((SUB, 128, 16), jnp.float32),
            pltpu.VMEM((6, CH), jnp.float32),
            pltpu.VMEM((6, CH), jnp.float32),
        ] + [pltpu.VMEM((_L16 * G,), jnp.float32)] * 5
          + [
            pltpu.VMEM((NB_SL,), jnp.int32),
            pltpu.VMEM((640,), jnp.float32),
            pltpu.SemaphoreType.DMA,
            pltpu.SemaphoreType.DMA,
            pltpu.SemaphoreType.DMA,
        ],
    )
    rec, sums = p1(full_edge_index[0], full_edge_index[1],
                   full_edge_attr[:, 0], table, batch)

    p2 = pl.kernel(
        _pass2_body,
        out_type=[jax.ShapeDtypeStruct((NW * G,), jnp.float32)],
        mesh=mesh,
        compiler_params=pltpu.CompilerParams(needs_layout_passes=False,
                                             use_tc_tiling_on_sc=False),
        scratch_types=[
            pltpu.VMEM((NW * 640,), jnp.float32),
            pltpu.VMEM((G,), jnp.float32),
            pltpu.VMEM((6, CH2), jnp.float32),
            pltpu.VMEM((6, CH2), jnp.float32),
            pltpu.VMEM((_L16 * G,), jnp.float32),
            pltpu.VMEM((G,), jnp.float32),
            pltpu.SemaphoreType.DMA,
        ],
    )
    stress, = p2(rec, sums)

    out = pl.pallas_call(
        _fin_body,
        out_shape=jax.ShapeDtypeStruct((1, 1), jnp.float32),
    )(sums.reshape(NW, 640), stress.reshape(NW, G))
    return out[0, 0]


# speedup vs baseline: 111.6351x; 1.0889x over previous
import functools

import jax
import jax.numpy as jnp
from jax import lax
from jax.experimental import pallas as pl
from jax.experimental.pallas import tpu as pltpu
from jax.experimental.pallas import tpu_sc as plsc

N = 100000
E = 6400000
G = 128
NW = 32
CH = 1024
SUB = CH // 128
NCHUNK = E // CH
TRIPS = -(-NCHUNK // NW) * NW // NW
PAIRS = TRIPS // 2
CH2 = 2048
NCHUNK2 = E // CH2
TRIPS2 = 100
QUADS2 = TRIPS2 // 4
NGRP2 = CH2 // 16
NB_SL = 3200
NNODE_IT = NB_SL // 16
NGRP = CH // 16

_L16 = 16


def _iota16():
    return lax.iota(jnp.int32, _L16)


def _rsqrt(x):
    bits = plsc.bitcast(x, jnp.int32)
    y = plsc.bitcast(jnp.full((_L16,), 0x5F3759DF, jnp.int32) - jnp.right_shift(bits, 1), jnp.float32)
    half = x * 0.5
    for _ in range(3):
        y = y * (1.5 - half * y * y)
    return y


def _recip(x):
    bits = plsc.bitcast(x, jnp.int32)
    y = plsc.bitcast(jnp.full((_L16,), 0x7EF311C3, jnp.int32) - bits, jnp.float32)
    for _ in range(3):
        y = y * (2.0 - x * y)
    return y


def _wid():
    return lax.axis_index("s") * 2 + lax.axis_index("c")


def _pass1_body(start_hbm, end_hbm, d_hbm, table_hbm, batch_hbm,
                rec_hbm, sums_hbm,
                sxa, exa, sxb, exb, dba, dbb,
                rsa, rea, rsb, reb,
                reca, recb,
                acc_num, acc_den, acc_ov, acc_cnt, acc_gsz,
                nbuf, stage, sem_in, sem_g, sem_o):
    wid = _wid()
    lane = _iota16()
    lane128 = lane * G
    zeros16 = jnp.zeros((_L16,), jnp.float32)
    ones16 = jnp.ones((_L16,), jnp.float32)
    c0 = jnp.zeros((_L16,), jnp.int32)
    c1 = jnp.full((_L16,), 1, jnp.int32)
    c2 = jnp.full((_L16,), 2, jnp.int32)
    c3 = jnp.full((_L16,), 3, jnp.int32)
    c4 = jnp.full((_L16,), 4, jnp.int32)

    @pl.loop(0, G)
    def _zero(i):
        sl = pl.ds(i * _L16, _L16)
        acc_num[sl] = zeros16
        acc_den[sl] = zeros16
        acc_ov[sl] = zeros16
        acc_cnt[sl] = zeros16
        acc_gsz[sl] = zeros16

    nbase = jnp.minimum(wid * NB_SL, N - NB_SL)
    pltpu.sync_copy(batch_hbm.at[pl.ds(nbase, NB_SL)], nbuf)

    @pl.loop(0, NNODE_IT)
    def _hist(t):
        v = nbuf[pl.ds(t * _L16, _L16)]
        gid = nbase + t * _L16 + lane
        mask = jnp.logical_and(gid >= wid * NB_SL, gid < N)
        plsc.addupdate_scatter(acc_gsz, [lane128 + v], ones16, mask=mask)

    def chunk_of(t):
        cid_raw = wid + NW * t
        valid = cid_raw < NCHUNK
        return jnp.where(valid, cid_raw, wid), valid

    def fire_in(cid, sx, ex, db):
        off = cid * CH
        pltpu.async_copy(d_hbm.at[pl.ds(off, CH)], db, sem_in)
        pltpu.async_copy(start_hbm.at[pl.ds(off, CH)], sx, sem_in)
        pltpu.async_copy(end_hbm.at[pl.ds(off, CH)], ex, sem_in)

    def wait_in(sx, ex, db):
        pltpu.make_async_copy(d_hbm.at[pl.ds(0, CH)], db, sem_in).wait()
        pltpu.make_async_copy(start_hbm.at[pl.ds(0, CH)], sx, sem_in).wait()
        pltpu.make_async_copy(end_hbm.at[pl.ds(0, CH)], ex, sem_in).wait()

    def fire_gather(sx, ex, rs, re):
        for k in range(SUB):
            pltpu.async_copy(table_hbm.at[sx.at[pl.ds(k * 128, 128)]], rs.at[k], sem_g)
            pltpu.async_copy(table_hbm.at[ex.at[pl.ds(k * 128, 128)]], re.at[k], sem_g)

    def wait_gather(rs, re):
        for k in range(SUB):
            pltpu.make_async_copy(table_hbm.at[pl.ds(0, 128)], rs.at[k], sem_g).wait()
            pltpu.make_async_copy(table_hbm.at[pl.ds(0, 128)], re.at[k], sem_g).wait()

    def fire_out(cid, bufs):
        off = cid * CH
        for q_i, b in enumerate(bufs):
            pltpu.async_copy(b, rec_hbm.at[q_i, pl.ds(off, CH)], sem_o)

    def wait_out(bufs):
        for q_i, b in enumerate(bufs):
            pltpu.make_async_copy(b, rec_hbm.at[q_i, pl.ds(0, CH)], sem_o).wait()

    def compute(valid, db, rs, re, bufs):
        limit = jnp.where(valid, _L16, 0)
        vmask = lane < limit
        r0, r1, r2, r3, r4, r5 = bufs

        @pl.loop(0, NGRP)
        def _grp(g):
            base = g * _L16
            ev = base + lane
            jv = jnp.right_shift(ev, 7)
            rv = jnp.bitwise_and(ev, 127)
            px_s = plsc.load_gather(rs, [jv, rv, c0])
            py_s = plsc.load_gather(rs, [jv, rv, c1])
            sx_s = plsc.load_gather(rs, [jv, rv, c2])
            sy_s = plsc.load_gather(rs, [jv, rv, c3])
            b_s = plsc.bitcast(plsc.load_gather(rs, [jv, rv, c4]), jnp.int32)
            px_e = plsc.load_gather(re, [jv, rv, c0])
            py_e = plsc.load_gather(re, [jv, rv, c1])
            sx_e = plsc.load_gather(re, [jv, rv, c2])
            sy_e = plsc.load_gather(re, [jv, rv, c3])
            b_e = plsc.bitcast(plsc.load_gather(re, [jv, rv, c4]), jnp.int32)
            d = db[pl.ds(base, _L16)]

            invd = _recip(d)
            dx = px_s - px_e
            dy = py_s - py_e
            q = dx * dx + dy * dy
            eu = q * _rsqrt(q)
            r = eu * invd
            binv = lane128 + b_s
            plsc.addupdate_scatter(acc_num, [binv], r * r, mask=vmask)
            plsc.addupdate_scatter(acc_den, [binv], r, mask=vmask)

            ox = jnp.maximum((sx_s + sx_e) * 0.5 - jnp.abs(dx), 0.0)
            oy = jnp.maximum((sy_s + sy_e) * 0.5 - jnp.abs(dy), 0.0)
            tot = sx_s + sy_s + sx_e + sy_e
            nov = ox * oy * _recip(tot)
            plsc.addupdate_scatter(acc_ov, [binv], nov, mask=vmask)
            plsc.addupdate_scatter(acc_cnt, [binv], ones16, mask=vmask)

            sl = pl.ds(base, _L16)
            r0[sl] = px_s
            r1[sl] = py_s
            r2[sl] = px_e
            r3[sl] = py_e
            r4[sl] = plsc.bitcast(jnp.left_shift(b_s, 7) + b_e, jnp.float32)
            r5[sl] = invd

    bufsA = (reca.at[0], reca.at[1], reca.at[2], reca.at[3], reca.at[4], reca.at[5])
    bufsB = (recb.at[0], recb.at[1], recb.at[2], recb.at[3], recb.at[4], recb.at[5])

    cid0, _ = chunk_of(0)
    fire_in(cid0, sxa, exa, dba)

    @pl.loop(0, PAIRS)
    def _pair(p):
        cidA, _va = chunk_of(2 * p)
        cidB, vB = chunk_of(2 * p + 1)
        wait_in(sxa, exa, dba)
        fire_gather(sxa, exa, rsa, rea)
        fire_in(cidB, sxb, exb, dbb)
        wait_gather(rsa, rea)

        @pl.when(p > 0)
        def _():
            wait_out(bufsA)
        compute(True, dba, rsa, rea, bufsA)
        fire_out(cidA, bufsA)
        wait_in(sxb, exb, dbb)
        fire_gather(sxb, exb, rsb, reb)

        @pl.when(p < PAIRS - 1)
        def _():
            cidA2, _ = chunk_of(2 * p + 2)
            fire_in(cidA2, sxa, exa, dba)
        wait_gather(rsb, reb)

        @pl.when(p > 0)
        def _():
            wait_out(bufsB)
        compute(vB, dbb, rsb, reb, bufsB)
        fire_out(cidB, bufsB)

    wait_out(bufsA)
    wait_out(bufsB)

    for q_i, acc in enumerate((acc_num, acc_den, acc_ov, acc_cnt, acc_gsz)):
        for blk in range(G // _L16):
            tot = acc[pl.ds(blk * _L16, _L16)]
            for l in range(1, _L16):
                tot = tot + acc[pl.ds(l * G + blk * _L16, _L16)]
            stage[pl.ds(q_i * G + blk * _L16, _L16)] = tot
    pltpu.sync_copy(stage, sums_hbm.at[pl.ds(wid * 640, 640)])


def _pass2_body(rec_hbm, sums_hbm, stress_hbm,
                sbuf, invs, rec0, rec1, rec2, rec3, acc_st, stage, sem_in):
    wid = _wid()
    lane = _iota16()
    lane128 = lane * G
    zeros16 = jnp.zeros((_L16,), jnp.float32)

    @pl.loop(0, G)
    def _zero(i):
        acc_st[pl.ds(i * _L16, _L16)] = zeros16

    pltpu.sync_copy(sums_hbm, sbuf)
    for blk in range(G // _L16):
        ntot = sbuf[pl.ds(blk * _L16, _L16)]
        dtot = sbuf[pl.ds(G + blk * _L16, _L16)]
        for w in range(1, NW):
            ntot = ntot + sbuf[pl.ds(w * 640 + blk * _L16, _L16)]
            dtot = dtot + sbuf[pl.ds(w * 640 + G + blk * _L16, _L16)]
        invs[pl.ds(blk * _L16, _L16)] = dtot * _recip(ntot)

    def chunk_of(t):
        cid_raw = wid + NW * t
        valid = cid_raw < NCHUNK2
        return jnp.where(valid, cid_raw, wid), valid

    def fire_rin(cid, bufs):
        off = cid * CH2
        for q_i in range(6):
            pltpu.async_copy(rec_hbm.at[q_i, pl.ds(off, CH2)], bufs.at[q_i], sem_in)

    def wait_rin(bufs):
        for q_i in range(6):
            pltpu.make_async_copy(rec_hbm.at[0, pl.ds(0, CH2)], bufs.at[q_i], sem_in).wait()

    def compute(valid, bufs):
        limit = jnp.where(valid, _L16, 0)
        vmask = lane < limit

        @pl.loop(0, NGRP2, unroll=4)
        def _grp(g):
            sl = pl.ds(g * _L16, _L16)
            px_s = bufs[0, sl]
            py_s = bufs[1, sl]
            px_e = bufs[2, sl]
            py_e = bufs[3, sl]
            bp = plsc.bitcast(bufs[4, sl], jnp.int32)
            invd = bufs[5, sl]
            b_s = jnp.right_shift(bp, 7)
            b_e = jnp.bitwise_and(bp, 127)
            u = plsc.load_gather(invs, [b_s])
            v = plsc.load_gather(invs, [b_e])
            ddx = px_s * u - px_e * v
            ddy = py_s * u - py_e * v
            q2 = ddx * ddx + ddy * ddy
            eu2 = q2 * _rsqrt(q2)
            t = eu2 * invd - 1.0
            plsc.addupdate_scatter(acc_st, [lane128 + b_s], t * t, mask=vmask)

    bufs4 = (rec0, rec1, rec2, rec3)
    for t in range(3):
        cid_t, _ = chunk_of(t)
        fire_rin(cid_t, bufs4[t])

    @pl.loop(0, QUADS2)
    def _quad(p):
        for i in range(4):
            tt = 4 * p + i
            cid_t, v_t = chunk_of(tt)
            wait_rin(bufs4[i])
            tf = tt + 3

            @pl.when(tf < TRIPS2)
            def _():
                cid_f, _ = chunk_of(tf)
                fire_rin(cid_f, bufs4[(i + 3) % 4])
            compute(v_t, bufs4[i])

    for blk in range(G // _L16):
        tot = acc_st[pl.ds(blk * _L16, _L16)]
        for l in range(1, _L16):
            tot = tot + acc_st[pl.ds(l * G + blk * _L16, _L16)]
        stage[pl.ds(blk * _L16, _L16)] = tot
    pltpu.sync_copy(stage, stress_hbm.at[pl.ds(wid * G, G)])


def _fin_body(sums_ref, stress_ref, o_ref):
    s = sums_ref[:]
    ov = jnp.sum(s[:, 2 * G:3 * G], axis=0)
    cnt = jnp.sum(s[:, 3 * G:4 * G], axis=0)
    gsz = jnp.sum(s[:, 4 * G:5 * G], axis=0)
    st = jnp.sum(stress_ref[:], axis=0)
    combined = st / (gsz * gsz) + ov / cnt
    o_ref[:, :] = jnp.mean(combined)[None, None]


@jax.jit
def kernel(node_pos, node_sizes, full_edge_index, batch, full_edge_attr):
    table = jnp.concatenate(
        [node_pos, node_sizes,
         lax.bitcast_convert_type(batch, jnp.float32)[:, None],
         jnp.zeros((N, 3), jnp.float32)], axis=1)

    mesh = plsc.VectorSubcoreMesh(core_axis_name="c", subcore_axis_name="s")

    p1 = pl.kernel(
        _pass1_body,
        out_type=[jax.ShapeDtypeStruct((6, E), jnp.float32),
                  jax.ShapeDtypeStruct((NW * 640,), jnp.float32)],
        mesh=mesh,
        compiler_params=pltpu.CompilerParams(needs_layout_passes=False,
                                             use_tc_tiling_on_sc=False),
        scratch_types=[
            pltpu.VMEM((CH,), jnp.int32),
            pltpu.VMEM((CH,), jnp.int32),
            pltpu.VMEM((CH,), jnp.int32),
            pltpu.VMEM((CH,), jnp.int32),
            pltpu.VMEM((CH,), jnp.float32),
            pltpu.VMEM((CH,), jnp.float32),
            pltpu.VMEM((SUB, 128, 8), jnp.float32),
            pltpu.VMEM((SUB, 128, 8), jnp.float32),
            pltpu.VMEM((SUB, 128, 8), jnp.float32),
            pltpu.VMEM((SUB, 128, 8), jnp.float32),
            pltpu.VMEM((6, CH), jnp.float32),
            pltpu.VMEM((6, CH), jnp.float32),
        ] + [pltpu.VMEM((_L16 * G,), jnp.float32)] * 5
          + [
            pltpu.VMEM((NB_SL,), jnp.int32),
            pltpu.VMEM((640,), jnp.float32),
            pltpu.SemaphoreType.DMA,
            pltpu.SemaphoreType.DMA,
            pltpu.SemaphoreType.DMA,
        ],
    )
    rec, sums = p1(full_edge_index[0], full_edge_index[1],
                   full_edge_attr[:, 0], table, batch)

    p2 = pl.kernel(
        _pass2_body,
        out_type=[jax.ShapeDtypeStruct((NW * G,), jnp.float32)],
        mesh=mesh,
        compiler_params=pltpu.CompilerParams(needs_layout_passes=False,
                                             use_tc_tiling_on_sc=False),
        scratch_types=[
            pltpu.VMEM((NW * 640,), jnp.float32),
            pltpu.VMEM((G,), jnp.float32),
            pltpu.VMEM((6, CH2), jnp.float32),
            pltpu.VMEM((6, CH2), jnp.float32),
            pltpu.VMEM((6, CH2), jnp.float32),
            pltpu.VMEM((6, CH2), jnp.float32),
            pltpu.VMEM((_L16 * G,), jnp.float32),
            pltpu.VMEM((G,), jnp.float32),
            pltpu.SemaphoreType.DMA,
        ],
    )
    stress, = p2(rec, sums)

    out = pl.pallas_call(
        _fin_body,
        out_shape=jax.ShapeDtypeStruct((1, 1), jnp.float32),
    )(sums.reshape(NW, 640), stress.reshape(NW, G))
    return out[0, 0]
